# baseline (speedup 1.0000x reference)
import jax
import jax.numpy as jnp
from jax import lax
from jax.experimental import pallas as pl
from jax.experimental.pallas import tpu as pltpu
from jax.experimental.pallas import tpu_sc as plsc

B_BITS = 15
NB = 1 << B_BITS
HIST = 2 * NB
N_IMG = 8
NPIX = 512 * 512
NC, NS = 2, 16
IMGS_PER_SC = N_IMG // NC
CHUNK = NPIX // NS
ROWS = CHUNK // 128
ZBUF = NB // 4
SC_HIST = IMGS_PER_SC * HIST
ZSLICE = SC_HIST // NS


def _sc_hist_body(lg_hbm, tg_hbm, cnt_out, sum_out,
                  lg_v, tg_v, idx_v, val_v, ones_v, zero_v, cnt_sh, sum_sh):
    c = lax.axis_index("c")
    s = lax.axis_index("s")

    def _zb(i, _):
        zero_v[pl.ds(i * 16, 16)] = jnp.zeros((16,), jnp.float32)
        return 0
    lax.fori_loop(0, ZBUF // 16, _zb, 0)

    def _ob(i, _):
        ones_v[pl.ds(i * 16, 16)] = jnp.ones((16,), jnp.float32)
        return 0
    lax.fori_loop(0, CHUNK // 16, _ob, 0)

    for k in range(ZSLICE // ZBUF):
        off = s * ZSLICE + k * ZBUF
        pltpu.sync_copy(zero_v, cnt_sh.at[pl.ds(off, ZBUF)])
        pltpu.sync_copy(zero_v, sum_sh.at[pl.ds(off, ZBUF)])
    plsc.subcore_barrier()

    for il in range(IMGS_PER_SC):
        img = c * IMGS_PER_SC + il
        base = img * NPIX + s * CHUNK
        pltpu.sync_copy(lg_hbm.at[pl.ds(base, CHUNK)], lg_v)
        pltpu.sync_copy(tg_hbm.at[pl.ds(base, CHUNK)], tg_v)

        def _row(row, _, il=il):
            for jj in range(8):
                i16 = (row * 8 + jj) * 16
                l = lg_v[pl.ds(i16, 16)]
                t = tg_v[pl.ds(i16, 16)]
                tb = t > 0.5
                sign = jnp.where(tb, 1.0, -1.0).astype(jnp.float32)
                r = jnp.maximum(1.0 - l * sign, 0.0).astype(jnp.float32)
                b = lax.shift_right_logical(
                    lax.bitcast_convert_type(r, jnp.int32), 32 - B_BITS)
                idx = b + jnp.where(tb, NB, 0).astype(jnp.int32) + il * HIST
                idx_v[pl.ds(i16, 16)] = idx
                val_v[pl.ds(i16, 16)] = r
            return 0
        lax.fori_loop(0, ROWS, _row, 0)

        pltpu.sync_copy(ones_v, cnt_sh.at[idx_v], add=True)
        pltpu.sync_copy(val_v, sum_sh.at[idx_v], add=True)

    plsc.subcore_barrier()
    off_sh = s * ZSLICE
    off_out = c * SC_HIST + s * ZSLICE
    pltpu.sync_copy(cnt_sh.at[pl.ds(off_sh, ZSLICE)],
                    cnt_out.at[pl.ds(off_out, ZSLICE)])
    pltpu.sync_copy(sum_sh.at[pl.ds(off_sh, ZSLICE)],
                    sum_out.at[pl.ds(off_out, ZSLICE)])


import functools


@functools.lru_cache(maxsize=None)
def _build_sc_hist():
  return pl.kernel(
    _sc_hist_body,
    out_type=(jax.ShapeDtypeStruct((N_IMG * HIST,), jnp.float32),
              jax.ShapeDtypeStruct((N_IMG * HIST,), jnp.float32)),
    mesh=plsc.VectorSubcoreMesh(core_axis_name="c", subcore_axis_name="s",
                                num_cores=NC, num_subcores=NS),
    scratch_types=[
        pltpu.VMEM((CHUNK,), jnp.float32),
        pltpu.VMEM((CHUNK,), jnp.float32),
        pltpu.VMEM((CHUNK,), jnp.int32),
        pltpu.VMEM((CHUNK,), jnp.float32),
        pltpu.VMEM((CHUNK,), jnp.float32),
        pltpu.VMEM((ZBUF,), jnp.float32),
        pltpu.VMEM_SHARED((SC_HIST,), jnp.float32),
        pltpu.VMEM_SHARED((SC_HIST,), jnp.float32),
    ],
  )


_R, _C = NB // 128, 128


def _suffix_excl(x, upper_incl, strict_lower):
    lane_cum = jax.lax.dot_general(
        x, upper_incl, (((1,), (0,)), ((), ())),
        preferred_element_type=jnp.float32)
    row_prev = jax.lax.dot_general(
        strict_lower, x, (((1,), (0,)), ((), ())),
        preferred_element_type=jnp.float32)
    prefix_incl = lane_cum + jnp.sum(row_prev, axis=1, keepdims=True)
    return jnp.sum(x) - prefix_incl


def _tc_finish_body(cnt_ref, sum_ref, out_ref):
    i = pl.program_id(0)

    @pl.when(i == 0)
    def _():
        out_ref[...] = jnp.zeros((1, 1), jnp.float32)

    cn = cnt_ref[0, 0]
    cp = cnt_ref[0, 1]
    sn = sum_ref[0, 0]
    sp = sum_ref[0, 1]

    ci = lax.broadcasted_iota(jnp.int32, (128, 128), 0)
    cj = lax.broadcasted_iota(jnp.int32, (128, 128), 1)
    upper_incl = (ci <= cj).astype(jnp.float32)
    ri = lax.broadcasted_iota(jnp.int32, (_R, _R), 0)
    rj = lax.broadcasted_iota(jnp.int32, (_R, _R), 1)
    strict_lower = (rj < ri).astype(jnp.float32)

    p = jnp.sum(cp)
    q_ab = _suffix_excl(cp, upper_incl, strict_lower)
    m_ab = _suffix_excl(cn, upper_incl, strict_lower)

    d0 = jnp.maximum(p + m_ab, 1.0)
    d1 = jnp.maximum(p + m_ab + cn, 1.0)
    pos_term = jnp.sum(sp / d0)
    coef = (p - q_ab - cp) * (1.0 / d0 - 1.0 / d1) / jnp.maximum(cn, 1.0)
    neg_term = jnp.sum(sn * coef)
    loss = pos_term + neg_term

    flat = (lax.broadcasted_iota(jnp.int32, (_R, _C), 0) * _C
            + lax.broadcasted_iota(jnp.int32, (_R, _C), 1))
    occupied = cn > 0.0
    bmax = jnp.max(jnp.where(occupied, flat, -1))
    loss0 = jnp.sum(jnp.where(flat == bmax, sn / jnp.maximum(cn, 1.0), 0.0))
    loss = jnp.where(p > 0.0, loss, loss0)

    out_ref[...] += (loss / N_IMG).reshape(1, 1)


_tc_finish = pl.pallas_call(
    _tc_finish_body,
    grid=(N_IMG,),
    in_specs=[
        pl.BlockSpec((1, 2, _R, _C), lambda i: (i, 0, 0, 0)),
        pl.BlockSpec((1, 2, _R, _C), lambda i: (i, 0, 0, 0)),
    ],
    out_specs=pl.BlockSpec((1, 1), lambda i: (0, 0)),
    out_shape=jax.ShapeDtypeStruct((1, 1), jnp.float32),
)


def kernel(logits, targets):
    lg = logits.reshape(N_IMG * NPIX)
    tg = targets.reshape(N_IMG * NPIX)
    cnt, sm = _build_sc_hist()(lg, tg)
    cnt = cnt.reshape(N_IMG, 2, _R, _C)
    sm = sm.reshape(N_IMG, 2, _R, _C)
    out = _tc_finish(cnt, sm)
    return out[0, 0]


# speedup vs baseline: 2.3244x; 2.3244x over previous
import functools

import jax
import jax.numpy as jnp
from jax import lax
from jax.experimental import pallas as pl
from jax.experimental.pallas import tpu as pltpu
from jax.experimental.pallas import tpu_sc as plsc

B_BITS = 12
NB = 1 << B_BITS
HIST = 2 * NB
N_IMG = 8
NPIX = 512 * 512
NC, NS = 2, 16
IMGS_PER_SC = N_IMG // NC
CHUNK = NPIX // NS
VPR = CHUNK // 16
UNROLL = 4


def _sc_hist_body(lg_hbm, tg_hbm, cnt_out, sum_out, mx_out,
                  lg_v, tg_v, lc, ls, mxv):
    c = lax.axis_index("c")
    s = lax.axis_index("s")

    for il in range(IMGS_PER_SC):
        img = c * IMGS_PER_SC + il
        base = img * NPIX + s * CHUNK
        pltpu.sync_copy(lg_hbm.at[pl.ds(base, CHUNK)], lg_v)
        pltpu.sync_copy(tg_hbm.at[pl.ds(base, CHUNK)], tg_v)

        def _zero(i, _):
            z = jnp.zeros((16,), jnp.float32)
            lc[pl.ds(i * 16, 16)] = z
            ls[pl.ds(i * 16, 16)] = z
            return 0
        lax.fori_loop(0, HIST // 16, _zero, 0)
        mxv[...] = jnp.zeros((16,), jnp.float32)

        def _vreg(i, _):
            for u in range(UNROLL):
                i16 = (i * UNROLL + u) * 16
                l = lg_v[pl.ds(i16, 16)]
                t = tg_v[pl.ds(i16, 16)]
                tb = t > 0.5
                sign = jnp.where(tb, 1.0, -1.0).astype(jnp.float32)
                r = jnp.maximum(1.0 - l * sign, 0.0).astype(jnp.float32)
                b = lax.shift_right_logical(
                    lax.bitcast_convert_type(r, jnp.int32), 32 - B_BITS)
                idx = b + jnp.where(tb, NB, 0).astype(jnp.int32)
                plsc.addupdate_scatter(lc, [idx], jnp.ones((16,), jnp.float32))
                plsc.addupdate_scatter(ls, [idx], r)
                mxv[...] = jnp.maximum(mxv[...], r)
            return 0
        lax.fori_loop(0, VPR // UNROLL, _vreg, 0)

        out_base = (img * NS + s) * HIST
        pltpu.sync_copy(lc, cnt_out.at[pl.ds(out_base, HIST)])
        pltpu.sync_copy(ls, sum_out.at[pl.ds(out_base, HIST)])
        pltpu.sync_copy(mxv, mx_out.at[pl.ds((img * NS + s) * 16, 16)])


@functools.lru_cache(maxsize=None)
def _build_sc_hist():
  return pl.kernel(
    _sc_hist_body,
    out_type=(jax.ShapeDtypeStruct((N_IMG * NS * HIST,), jnp.float32),
              jax.ShapeDtypeStruct((N_IMG * NS * HIST,), jnp.float32),
              jax.ShapeDtypeStruct((N_IMG * NS * 16,), jnp.float32)),
    mesh=plsc.VectorSubcoreMesh(core_axis_name="c", subcore_axis_name="s",
                                num_cores=NC, num_subcores=NS),
    compiler_params=pltpu.CompilerParams(needs_layout_passes=False),
    scratch_types=[
        pltpu.VMEM((CHUNK,), jnp.float32),
        pltpu.VMEM((CHUNK,), jnp.float32),
        pltpu.VMEM((HIST,), jnp.float32),
        pltpu.VMEM((HIST,), jnp.float32),
        pltpu.VMEM((16,), jnp.float32),
    ],
  )


_R, _C = NB // 128, 128


def _suffix_excl(x, upper_incl, strict_lower):
    lane_cum = jax.lax.dot_general(
        x, upper_incl, (((1,), (0,)), ((), ())),
        preferred_element_type=jnp.float32)
    row_prev = jax.lax.dot_general(
        strict_lower, x, (((1,), (0,)), ((), ())),
        preferred_element_type=jnp.float32)
    prefix_incl = lane_cum + jnp.sum(row_prev, axis=1, keepdims=True)
    return jnp.sum(x) - prefix_incl


def _tc_finish_body(cnt_ref, sum_ref, mx_ref, out_ref):
    i = pl.program_id(0)

    @pl.when(i == 0)
    def _():
        out_ref[...] = jnp.zeros((1, 1), jnp.float32)

    cnt = jnp.sum(cnt_ref[0], axis=0)
    sm = jnp.sum(sum_ref[0], axis=0)
    cn, cp = cnt[0], cnt[1]
    sn, sp = sm[0], sm[1]

    ci = lax.broadcasted_iota(jnp.int32, (128, 128), 0)
    cj = lax.broadcasted_iota(jnp.int32, (128, 128), 1)
    upper_incl = (ci <= cj).astype(jnp.float32)
    ri = lax.broadcasted_iota(jnp.int32, (_R, _R), 0)
    rj = lax.broadcasted_iota(jnp.int32, (_R, _R), 1)
    strict_lower = (rj < ri).astype(jnp.float32)

    p = jnp.sum(cp)
    q_ab = _suffix_excl(cp, upper_incl, strict_lower)
    m_ab = _suffix_excl(cn, upper_incl, strict_lower)

    d0 = jnp.maximum(p + m_ab, 1.0)
    d1 = jnp.maximum(p + m_ab + cn, 1.0)
    pos_term = jnp.sum(sp / d0)
    coef = (p - q_ab - cp) * (1.0 / d0 - 1.0 / d1) / jnp.maximum(cn, 1.0)
    neg_term = jnp.sum(sn * coef)
    loss = pos_term + neg_term

    loss0 = jnp.max(mx_ref[0])
    loss = jnp.where(p > 0.0, loss, loss0)

    out_ref[...] += (loss / N_IMG).reshape(1, 1)


_tc_finish = pl.pallas_call(
    _tc_finish_body,
    grid=(N_IMG,),
    in_specs=[
        pl.BlockSpec((1, NS, 2, _R, _C), lambda i: (i, 0, 0, 0, 0)),
        pl.BlockSpec((1, NS, 2, _R, _C), lambda i: (i, 0, 0, 0, 0)),
        pl.BlockSpec((1, NS, 16), lambda i: (i, 0, 0)),
    ],
    out_specs=pl.BlockSpec((1, 1), lambda i: (0, 0)),
    out_shape=jax.ShapeDtypeStruct((1, 1), jnp.float32),
)


def kernel(logits, targets):
    lg = logits.reshape(N_IMG * NPIX)
    tg = targets.reshape(N_IMG * NPIX)
    cnt, sm, mx = _build_sc_hist()(lg, tg)
    cnt = cnt.reshape(N_IMG, NS, 2, _R, _C)
    sm = sm.reshape(N_IMG, NS, 2, _R, _C)
    mx = mx.reshape(N_IMG, NS, 16)
    out = _tc_finish(cnt, sm, mx)
    return out[0, 0]


# speedup vs baseline: 3.8958x; 1.6760x over previous
import functools

import jax
import jax.numpy as jnp
from jax import lax
from jax.experimental import pallas as pl
from jax.experimental.pallas import tpu as pltpu
from jax.experimental.pallas import tpu_sc as plsc

B_BITS = 12
NB = 1 << B_BITS
HIST = 2 * NB
N_IMG = 8
NPIX = 512 * 512
NC, NS = 2, 16
IMGS_PER_SC = N_IMG // NC
CHUNK = NPIX // NS
VPR = CHUNK // 16
UNROLL = 4


def _sc_hist_body(lg_hbm, tg_hbm, cnt_out, sum_out, mx_out,
                  lg_v, tg_v, lc, ls, mxv):
    c = lax.axis_index("c")
    s = lax.axis_index("s")

    @plsc.parallel_loop(0, IMGS_PER_SC * HIST // 16, unroll=8)
    def _zero(i):
        z = jnp.zeros((16,), jnp.float32)
        lc[pl.ds(i * 16, 16)] = z
        ls[pl.ds(i * 16, 16)] = z

    for il in range(IMGS_PER_SC):
        img = c * IMGS_PER_SC + il
        base = img * NPIX + s * CHUNK
        pltpu.sync_copy(lg_hbm.at[pl.ds(base, CHUNK)], lg_v)
        pltpu.sync_copy(tg_hbm.at[pl.ds(base, CHUNK)], tg_v)

        @plsc.parallel_loop(0, VPR, unroll=8,
                            carry=jnp.zeros((16,), jnp.float32))
        def _vreg(i, mxc, il=il):
            i16 = i * 16
            l = lg_v[pl.ds(i16, 16)]
            t = tg_v[pl.ds(i16, 16)]
            tb = t > 0.5
            sign = jnp.where(tb, 1.0, -1.0).astype(jnp.float32)
            r = jnp.maximum(1.0 - l * sign, 0.0).astype(jnp.float32)
            b = lax.shift_right_logical(
                lax.bitcast_convert_type(r, jnp.int32), 32 - B_BITS)
            idx = b + jnp.where(tb, NB, 0).astype(jnp.int32) + il * HIST
            plsc.addupdate_scatter(lc, [idx], jnp.ones((16,), jnp.float32))
            plsc.addupdate_scatter(ls, [idx], r)
            return jnp.maximum(mxc, r)
        mxv[pl.ds(il * 16, 16)] = _vreg

    for il in range(IMGS_PER_SC):
        img = c * IMGS_PER_SC + il
        out_base = (img * NS + s) * HIST
        pltpu.sync_copy(lc.at[pl.ds(il * HIST, HIST)],
                        cnt_out.at[pl.ds(out_base, HIST)])
        pltpu.sync_copy(ls.at[pl.ds(il * HIST, HIST)],
                        sum_out.at[pl.ds(out_base, HIST)])
        pltpu.sync_copy(mxv.at[pl.ds(il * 16, 16)],
                        mx_out.at[pl.ds((img * NS + s) * 16, 16)])


@functools.lru_cache(maxsize=None)
def _build_sc_hist():
  return pl.kernel(
    _sc_hist_body,
    out_type=(jax.ShapeDtypeStruct((N_IMG * NS * HIST,), jnp.float32),
              jax.ShapeDtypeStruct((N_IMG * NS * HIST,), jnp.float32),
              jax.ShapeDtypeStruct((N_IMG * NS * 16,), jnp.float32)),
    mesh=plsc.VectorSubcoreMesh(core_axis_name="c", subcore_axis_name="s",
                                num_cores=NC, num_subcores=NS),
    compiler_params=pltpu.CompilerParams(needs_layout_passes=False),
    scratch_types=[
        pltpu.VMEM((CHUNK,), jnp.float32),
        pltpu.VMEM((CHUNK,), jnp.float32),
        pltpu.VMEM((IMGS_PER_SC * HIST,), jnp.float32),
        pltpu.VMEM((IMGS_PER_SC * HIST,), jnp.float32),
        pltpu.VMEM((IMGS_PER_SC * 16,), jnp.float32),
    ],
  )


_R, _C = NB // 128, 128


def _suffix_excl(x, upper_incl, strict_lower):
    lane_cum = jax.lax.dot_general(
        x, upper_incl, (((1,), (0,)), ((), ())),
        preferred_element_type=jnp.float32)
    row_prev = jax.lax.dot_general(
        strict_lower, x, (((1,), (0,)), ((), ())),
        preferred_element_type=jnp.float32)
    prefix_incl = lane_cum + jnp.sum(row_prev, axis=1, keepdims=True)
    return jnp.sum(x) - prefix_incl


def _tc_finish_body(cnt_ref, sum_ref, mx_ref, out_ref):
    i = pl.program_id(0)

    @pl.when(i == 0)
    def _():
        out_ref[...] = jnp.zeros((1, 1), jnp.float32)

    cnt = jnp.sum(cnt_ref[0], axis=0)
    sm = jnp.sum(sum_ref[0], axis=0)
    cn, cp = cnt[0], cnt[1]
    sn, sp = sm[0], sm[1]

    ci = lax.broadcasted_iota(jnp.int32, (128, 128), 0)
    cj = lax.broadcasted_iota(jnp.int32, (128, 128), 1)
    upper_incl = (ci <= cj).astype(jnp.float32)
    ri = lax.broadcasted_iota(jnp.int32, (_R, _R), 0)
    rj = lax.broadcasted_iota(jnp.int32, (_R, _R), 1)
    strict_lower = (rj < ri).astype(jnp.float32)

    p = jnp.sum(cp)
    q_ab = _suffix_excl(cp, upper_incl, strict_lower)
    m_ab = _suffix_excl(cn, upper_incl, strict_lower)

    d0 = jnp.maximum(p + m_ab, 1.0)
    d1 = jnp.maximum(p + m_ab + cn, 1.0)
    pos_term = jnp.sum(sp / d0)
    coef = (p - q_ab - cp) * (1.0 / d0 - 1.0 / d1) / jnp.maximum(cn, 1.0)
    neg_term = jnp.sum(sn * coef)
    loss = pos_term + neg_term

    loss0 = jnp.max(mx_ref[0])
    loss = jnp.where(p > 0.0, loss, loss0)

    out_ref[...] += (loss / N_IMG).reshape(1, 1)


_tc_finish = pl.pallas_call(
    _tc_finish_body,
    grid=(N_IMG,),
    in_specs=[
        pl.BlockSpec((1, NS, 2, _R, _C), lambda i: (i, 0, 0, 0, 0)),
        pl.BlockSpec((1, NS, 2, _R, _C), lambda i: (i, 0, 0, 0, 0)),
        pl.BlockSpec((1, NS, 16), lambda i: (i, 0, 0)),
    ],
    out_specs=pl.BlockSpec((1, 1), lambda i: (0, 0)),
    out_shape=jax.ShapeDtypeStruct((1, 1), jnp.float32),
)


def kernel(logits, targets):
    lg = logits.reshape(N_IMG * NPIX)
    tg = targets.reshape(N_IMG * NPIX)
    cnt, sm, mx = _build_sc_hist()(lg, tg)
    cnt = cnt.reshape(N_IMG, NS, 2, _R, _C)
    sm = sm.reshape(N_IMG, NS, 2, _R, _C)
    mx = mx.reshape(N_IMG, NS, 16)
    out = _tc_finish(cnt, sm, mx)
    return out[0, 0]


# speedup vs baseline: 4.2990x; 1.1035x over previous
import functools

import jax
import jax.numpy as jnp
from jax import lax
from jax.experimental import pallas as pl
from jax.experimental.pallas import tpu as pltpu
from jax.experimental.pallas import tpu_sc as plsc

B_BITS = 12
NB = 1 << B_BITS
HIST = 2 * NB
N_IMG = 8
NPIX = 512 * 512
NC, NS = 2, 16
IMGS_PER_SC = N_IMG // NC
CHUNK = NPIX // NS
VPR = CHUNK // 16
UNROLL = 4


HALF = CHUNK // 2


def _sc_hist_body(lg_hbm, tg_hbm, cnt_out, sum_out, mx_out,
                  lgA, tgA, lgB, tgB, lc, ls, mxv, semL, semT):
    c = lax.axis_index("c")
    s = lax.axis_index("s")

    @plsc.parallel_loop(0, IMGS_PER_SC * HIST // 16, unroll=8)
    def _zero(i):
        z = jnp.zeros((16,), jnp.float32)
        lc[pl.ds(i * 16, 16)] = z
        ls[pl.ds(i * 16, 16)] = z

    steps = [(il, sub) for il in range(IMGS_PER_SC) for sub in range(2)]
    bufs = [(lgA, tgA), (lgB, tgB)]

    def _start(j):
        il, sub = steps[j]
        base = (c * IMGS_PER_SC + il) * NPIX + s * CHUNK + sub * HALF
        lgb, tgb = bufs[j % 2]
        return (pltpu.async_copy(lg_hbm.at[pl.ds(base, HALF)], lgb, semL),
                pltpu.async_copy(tg_hbm.at[pl.ds(base, HALF)], tgb, semT))

    hs = _start(0)
    mx_prev = jnp.zeros((16,), jnp.float32)
    for j, (il, sub) in enumerate(steps):
        lgb, tgb = bufs[j % 2]
        hs[0].wait()
        hs[1].wait()
        if j + 1 < len(steps):
            hs = _start(j + 1)
        carry0 = jnp.zeros((16,), jnp.float32) if sub == 0 else mx_prev

        @plsc.parallel_loop(0, HALF // 16, unroll=8, carry=carry0)
        def _vreg(i, mxc, il=il, lgb=lgb, tgb=tgb):
            i16 = i * 16
            l = lgb[pl.ds(i16, 16)]
            t = tgb[pl.ds(i16, 16)]
            tb = t > 0.5
            sign = jnp.where(tb, 1.0, -1.0).astype(jnp.float32)
            r = jnp.maximum(1.0 - l * sign, 0.0).astype(jnp.float32)
            b = lax.shift_right_logical(
                lax.bitcast_convert_type(r, jnp.int32), 32 - B_BITS)
            idx = b + jnp.where(tb, NB, 0).astype(jnp.int32) + il * HIST
            plsc.addupdate_scatter(lc, [idx], jnp.ones((16,), jnp.float32))
            plsc.addupdate_scatter(ls, [idx], r)
            return jnp.maximum(mxc, r)
        mx_prev = _vreg
        if sub == 1:
            mxv[pl.ds(il * 16, 16)] = mx_prev

    for il in range(IMGS_PER_SC):
        img = c * IMGS_PER_SC + il
        out_base = (img * NS + s) * HIST
        pltpu.sync_copy(lc.at[pl.ds(il * HIST, HIST)],
                        cnt_out.at[pl.ds(out_base, HIST)])
        pltpu.sync_copy(ls.at[pl.ds(il * HIST, HIST)],
                        sum_out.at[pl.ds(out_base, HIST)])
        pltpu.sync_copy(mxv.at[pl.ds(il * 16, 16)],
                        mx_out.at[pl.ds((img * NS + s) * 16, 16)])


@functools.lru_cache(maxsize=None)
def _build_sc_hist():
  return pl.kernel(
    _sc_hist_body,
    out_type=(jax.ShapeDtypeStruct((N_IMG * NS * HIST,), jnp.float32),
              jax.ShapeDtypeStruct((N_IMG * NS * HIST,), jnp.float32),
              jax.ShapeDtypeStruct((N_IMG * NS * 16,), jnp.float32)),
    mesh=plsc.VectorSubcoreMesh(core_axis_name="c", subcore_axis_name="s",
                                num_cores=NC, num_subcores=NS),
    compiler_params=pltpu.CompilerParams(needs_layout_passes=False),
    scratch_types=[
        pltpu.VMEM((HALF,), jnp.float32),
        pltpu.VMEM((HALF,), jnp.float32),
        pltpu.VMEM((HALF,), jnp.float32),
        pltpu.VMEM((HALF,), jnp.float32),
        pltpu.VMEM((IMGS_PER_SC * HIST,), jnp.float32),
        pltpu.VMEM((IMGS_PER_SC * HIST,), jnp.float32),
        pltpu.VMEM((IMGS_PER_SC * 16,), jnp.float32),
        pltpu.SemaphoreType.DMA,
        pltpu.SemaphoreType.DMA,
    ],
  )


_R, _C = NB // 128, 128


def _suffix_excl(x, upper_incl, strict_lower):
    lane_cum = jax.lax.dot_general(
        x, upper_incl, (((1,), (0,)), ((), ())),
        preferred_element_type=jnp.float32)
    row_prev = jax.lax.dot_general(
        strict_lower, x, (((1,), (0,)), ((), ())),
        preferred_element_type=jnp.float32)
    prefix_incl = lane_cum + jnp.sum(row_prev, axis=1, keepdims=True)
    return jnp.sum(x) - prefix_incl


def _tc_finish_body(cnt_ref, sum_ref, mx_ref, out_ref):
    i = pl.program_id(0)

    @pl.when(i == 0)
    def _():
        out_ref[...] = jnp.zeros((1, 1), jnp.float32)

    cnt = jnp.sum(cnt_ref[0], axis=0)
    sm = jnp.sum(sum_ref[0], axis=0)
    cn, cp = cnt[0], cnt[1]
    sn, sp = sm[0], sm[1]

    ci = lax.broadcasted_iota(jnp.int32, (128, 128), 0)
    cj = lax.broadcasted_iota(jnp.int32, (128, 128), 1)
    upper_incl = (ci <= cj).astype(jnp.float32)
    ri = lax.broadcasted_iota(jnp.int32, (_R, _R), 0)
    rj = lax.broadcasted_iota(jnp.int32, (_R, _R), 1)
    strict_lower = (rj < ri).astype(jnp.float32)

    p = jnp.sum(cp)
    q_ab = _suffix_excl(cp, upper_incl, strict_lower)
    m_ab = _suffix_excl(cn, upper_incl, strict_lower)

    d0 = jnp.maximum(p + m_ab, 1.0)
    d1 = jnp.maximum(p + m_ab + cn, 1.0)
    pos_term = jnp.sum(sp / d0)
    coef = (p - q_ab - cp) * (1.0 / d0 - 1.0 / d1) / jnp.maximum(cn, 1.0)
    neg_term = jnp.sum(sn * coef)
    loss = pos_term + neg_term

    loss0 = jnp.max(mx_ref[0])
    loss = jnp.where(p > 0.0, loss, loss0)

    out_ref[...] += (loss / N_IMG).reshape(1, 1)


_tc_finish = pl.pallas_call(
    _tc_finish_body,
    grid=(N_IMG,),
    in_specs=[
        pl.BlockSpec((1, NS, 2, _R, _C), lambda i: (i, 0, 0, 0, 0)),
        pl.BlockSpec((1, NS, 2, _R, _C), lambda i: (i, 0, 0, 0, 0)),
        pl.BlockSpec((1, NS, 16), lambda i: (i, 0, 0)),
    ],
    out_specs=pl.BlockSpec((1, 1), lambda i: (0, 0)),
    out_shape=jax.ShapeDtypeStruct((1, 1), jnp.float32),
)


def kernel(logits, targets):
    lg = logits.reshape(N_IMG * NPIX)
    tg = targets.reshape(N_IMG * NPIX)
    cnt, sm, mx = _build_sc_hist()(lg, tg)
    cnt = cnt.reshape(N_IMG, NS, 2, _R, _C)
    sm = sm.reshape(N_IMG, NS, 2, _R, _C)
    mx = mx.reshape(N_IMG, NS, 16)
    out = _tc_finish(cnt, sm, mx)
    return out[0, 0]
